# baseline (speedup 1.0000x reference)
import functools

import jax
import jax.numpy as jnp
from jax import lax
from jax.experimental import pallas as pl
from jax.experimental.pallas import tpu as pltpu
from jax.experimental.pallas import tpu_sc as plsc

_LANES = 16


def _linearize_table(tableT):
    E, V = tableT.shape
    bc = 4096
    hb = bc // 2
    nbf = V // bc
    rem = V - nbf * bc
    nb = nbf + (1 if rem else 0)
    out_sds = jax.ShapeDtypeStruct((nb * hb, 2 * E), jnp.float32)

    def body(tt_ref, o_ref):
        e_ids = jax.lax.broadcasted_iota(jnp.int32, (E, 2 * E), 0)
        q_ids = jax.lax.broadcasted_iota(jnp.int32, (E, 2 * E), 1)
        g1 = (q_ids == e_ids).astype(jnp.float32)
        g2 = (q_ids == e_ids + E).astype(jnp.float32)
        dg = (((0,), (0,)), ((), ()))
        o_ref[...] = (
            lax.dot_general(tt_ref[:, :hb], g1, dg,
                            preferred_element_type=jnp.float32)
            + lax.dot_general(tt_ref[:, hb:], g2, dg,
                              preferred_element_type=jnp.float32))

    main = pl.pallas_call(
        body,
        grid=(nbf,),
        in_specs=[pl.BlockSpec((E, bc), lambda c: (0, c))],
        out_specs=pl.BlockSpec((hb, 2 * E), lambda c: (c, 0)),
        out_shape=out_sds,
    )(tableT)
    if not rem:
        return main

    tail = jnp.pad(lax.slice(tableT, (0, nbf * bc), (E, V)),
                   ((0, 0), (0, bc - rem)))

    def body2(prev_ref, tt_ref, o_ref):
        del prev_ref
        body(tt_ref, o_ref)

    return pl.pallas_call(
        body2,
        grid=(1,),
        in_specs=[
            pl.BlockSpec(memory_space=pl.MemorySpace.ANY),
            pl.BlockSpec((E, bc), lambda c: (0, 0)),
        ],
        out_specs=pl.BlockSpec((hb, 2 * E), lambda c: (nbf, 0)),
        out_shape=out_sds,
        input_output_aliases={0: 0},
    )(main, tail)


@functools.partial(jax.jit, static_argnames=("nchunk", "k"))
def _pool(idx2, table, *, nchunk, k):
    B, L = idx2.shape
    assert L == nchunk * k
    E = table.shape[1]

    info = plsc.get_sparse_core_info()
    nw = info.num_cores * info.num_subcores
    bpw = B // nw
    ncol = E // _LANES
    runroll = 8

    mesh = plsc.VectorSubcoreMesh(core_axis_name="c", subcore_axis_name="s")

    nbuf = 4

    @functools.partial(
        pl.kernel,
        out_type=jax.ShapeDtypeStruct((B, E), jnp.float32),
        mesh=mesh,
        scratch_types=[
            pltpu.VMEM((bpw, L), jnp.int32),
            pltpu.VMEM((nbuf, L, E), jnp.float32),
            pltpu.VMEM((bpw, E), jnp.float32),
        ] + [pltpu.SemaphoreType.DMA] * nbuf,
        compiler_params=pltpu.CompilerParams(use_tc_tiling_on_sc=False),
        cost_estimate=pl.CostEstimate(
            flops=2 * B * L * E, transcendentals=0,
            bytes_accessed=B * L * (E * 4 + 4) + B * E * 4),
    )
    def pool(texts_hbm, table_hbm, out_hbm, idx_v, rows_v, pooled_v,
             *sems):
        wid = lax.axis_index("s") * info.num_cores + lax.axis_index("c")
        base = wid * bpw

        pltpu.sync_copy(texts_hbm.at[pl.ds(base, bpw)], idx_v)

        chunks = ((0, 104), (104, L - 104)) if L > 104 else ((0, L),)

        def gather_bag(b, u):
            for (o, n) in chunks:
                pltpu.async_copy(
                    table_hbm.at[idx_v.at[b, pl.ds(o, n)]],
                    rows_v.at[u, pl.ds(o, n)],
                    sems[u],
                )

        def drain_bag(b, u):
            for (o, n) in chunks:
                pltpu.make_async_copy(
                    table_hbm.at[idx_v.at[b, pl.ds(o, n)]],
                    rows_v.at[u, pl.ds(o, n)],
                    sems[u],
                ).wait()

        inv_l = jnp.float32(1.0 / L)

        def reduce_bag(b, u):
            def rbody(r, accs):
                out = list(accs)
                for v in range(runroll):
                    row = r * runroll + v
                    for c in range(ncol):
                        out[c] = out[c] + rows_v[u, row,
                                                 pl.ds(c * _LANES, _LANES)]
                return tuple(out)

            zero = jnp.zeros((_LANES,), jnp.float32)
            accs = lax.fori_loop(0, L // runroll, rbody, (zero,) * ncol)
            for c in range(ncol):
                pooled_v[b, pl.ds(c * _LANES, _LANES)] = accs[c] * inv_l

        for u in range(nbuf - 1):
            gather_bag(u, u)

        def body(q, carry):
            bq = nbuf * q
            for u in range(nbuf):
                b = bq + u
                nxt = b + nbuf - 1

                @pl.when(nxt < bpw)
                def _():
                    gather_bag(nxt, (u + nbuf - 1) % nbuf)

                drain_bag(b, u)
                reduce_bag(b, u)
            return carry

        lax.fori_loop(0, bpw // nbuf, body, 0)

        pltpu.sync_copy(pooled_v, out_hbm.at[pl.ds(base, bpw)])

    return pool(idx2, table)


def _mlp_chunk(prev, pooled, W1T, b1r, W2T, b2r, *, s, NB):
    Bs, E = pooled.shape
    H = W1T.shape[0]
    C = W2T.shape[0]
    cb = 600
    grid = (pl.cdiv(C, cb),)

    def compute(p_ref, w1t_ref, b1_ref, w2t_ref, b2_ref, ot_ref):
        ht = jax.lax.dot_general(
            w1t_ref[...], p_ref[...], (((1,), (1,)), ((), ())),
            preferred_element_type=jnp.float32) + b1_ref[...]
        ht = jnp.maximum(ht, 0.0)
        ot = jnp.dot(w2t_ref[...], ht, preferred_element_type=jnp.float32)
        ot_ref[...] = ot + b2_ref[...]

    in_specs = [
        pl.BlockSpec((Bs, E), lambda j: (0, 0)),
        pl.BlockSpec((H, E), lambda j: (0, 0)),
        pl.BlockSpec((H, 1), lambda j: (0, 0)),
        pl.BlockSpec((cb, H), lambda j: (j, 0)),
        pl.BlockSpec((cb, 1), lambda j: (j, 0)),
    ]
    args = [pooled, W1T, b1r, W2T, b2r]
    kwargs = {}
    if prev is None:
        body = compute
    else:
        def body(prev_ref, *refs):
            del prev_ref
            compute(*refs)
        in_specs = [pl.BlockSpec(memory_space=pl.MemorySpace.ANY)] + in_specs
        args = [prev] + args
        kwargs = dict(input_output_aliases={0: 0})

    return pl.pallas_call(
        body,
        grid=grid,
        in_specs=in_specs,
        out_specs=pl.BlockSpec((cb, Bs), lambda j, s=s: (j, s)),
        out_shape=jax.ShapeDtypeStruct((C, NB), jnp.float32),
        **kwargs,
    )(*args)


def kernel(texts, table, W1, b1, W2, b2):
    B, L = texts.shape
    V, E = table.shape
    nchunk = 5
    k = L // nchunk
    packed = _linearize_table(table.T)
    rows2 = packed.shape[0] * 2
    table_p = packed.reshape(rows2, E)
    ti = texts.astype(jnp.int32)
    g_idx = ((ti >> 12) << 12) | ((ti & 2047) << 1) | ((ti >> 11) & 1)

    H = W1.shape[1]
    C = W2.shape[1]
    W1T, W2T = W1.T, W2.T
    b1r, b2r = b1.reshape(H, 1), b2.reshape(C, 1)
    nsplit = 2
    bs = B // nsplit
    outT = None
    for s in range(nsplit):
        pooled_s = _pool(g_idx[s * bs:(s + 1) * bs], table_p,
                         nchunk=nchunk, k=k)
        outT = _mlp_chunk(outT, pooled_s, W1T, b1r, W2T, b2r, s=s, NB=B)
    return outT.T


# speedup vs baseline: 1.0514x; 1.0514x over previous
import functools

import jax
import jax.numpy as jnp
from jax import lax
from jax.experimental import pallas as pl
from jax.experimental.pallas import tpu as pltpu
from jax.experimental.pallas import tpu_sc as plsc

_LANES = 16


def _linearize_table(tableT):
    E, V = tableT.shape
    bc = 4096
    hb = bc // 2
    nbf = V // bc
    rem = V - nbf * bc
    nb = nbf + (1 if rem else 0)
    out_sds = jax.ShapeDtypeStruct((nb * hb, 2 * E), jnp.float32)

    def body(tt_ref, o_ref):
        e_ids = jax.lax.broadcasted_iota(jnp.int32, (E, 2 * E), 0)
        q_ids = jax.lax.broadcasted_iota(jnp.int32, (E, 2 * E), 1)
        g1 = (q_ids == e_ids).astype(jnp.float32)
        g2 = (q_ids == e_ids + E).astype(jnp.float32)
        dg = (((0,), (0,)), ((), ()))
        o_ref[...] = (
            lax.dot_general(tt_ref[:, :hb], g1, dg,
                            preferred_element_type=jnp.float32)
            + lax.dot_general(tt_ref[:, hb:], g2, dg,
                              preferred_element_type=jnp.float32))

    main = pl.pallas_call(
        body,
        grid=(nbf,),
        in_specs=[pl.BlockSpec((E, bc), lambda c: (0, c))],
        out_specs=pl.BlockSpec((hb, 2 * E), lambda c: (c, 0)),
        out_shape=out_sds,
    )(tableT)
    if not rem:
        return main

    tail = jnp.pad(lax.slice(tableT, (0, nbf * bc), (E, V)),
                   ((0, 0), (0, bc - rem)))

    def body2(prev_ref, tt_ref, o_ref):
        del prev_ref
        body(tt_ref, o_ref)

    return pl.pallas_call(
        body2,
        grid=(1,),
        in_specs=[
            pl.BlockSpec(memory_space=pl.MemorySpace.ANY),
            pl.BlockSpec((E, bc), lambda c: (0, 0)),
        ],
        out_specs=pl.BlockSpec((hb, 2 * E), lambda c: (nbf, 0)),
        out_shape=out_sds,
        input_output_aliases={0: 0},
    )(main, tail)


@functools.partial(jax.jit, static_argnames=("nchunk", "k"))
def _pool(idx2, table, *, nchunk, k):
    B, L = idx2.shape
    assert L == nchunk * k
    E = table.shape[1]

    info = plsc.get_sparse_core_info()
    nw = info.num_cores * info.num_subcores
    bpw = B // nw
    ncol = E // _LANES
    runroll = 8

    mesh = plsc.VectorSubcoreMesh(core_axis_name="c", subcore_axis_name="s")

    nbuf = 4

    @functools.partial(
        pl.kernel,
        out_type=jax.ShapeDtypeStruct((B, E), jnp.float32),
        mesh=mesh,
        scratch_types=[
            pltpu.VMEM((bpw, L), jnp.int32),
            pltpu.VMEM((nbuf, L, E), jnp.float32),
            pltpu.VMEM((bpw, E), jnp.float32),
        ] + [pltpu.SemaphoreType.DMA] * nbuf,
        compiler_params=pltpu.CompilerParams(use_tc_tiling_on_sc=False),
    )
    def pool(texts_hbm, table_hbm, out_hbm, idx_v, rows_v, pooled_v,
             *sems):
        wid = lax.axis_index("s") * info.num_cores + lax.axis_index("c")
        base = wid * bpw

        pltpu.sync_copy(texts_hbm.at[pl.ds(base, bpw)], idx_v)

        chunks = ((0, 104), (104, L - 104)) if L > 104 else ((0, L),)

        def gather_bag(b, u):
            for (o, n) in chunks:
                pltpu.async_copy(
                    table_hbm.at[idx_v.at[b, pl.ds(o, n)]],
                    rows_v.at[u, pl.ds(o, n)],
                    sems[u],
                )

        def drain_bag(b, u):
            for (o, n) in chunks:
                pltpu.make_async_copy(
                    table_hbm.at[idx_v.at[b, pl.ds(o, n)]],
                    rows_v.at[u, pl.ds(o, n)],
                    sems[u],
                ).wait()

        inv_l = jnp.float32(1.0 / L)

        def reduce_bag(b, u):
            def rbody(r, accs):
                out = list(accs)
                for v in range(runroll):
                    row = r * runroll + v
                    for c in range(ncol):
                        out[c] = out[c] + rows_v[u, row,
                                                 pl.ds(c * _LANES, _LANES)]
                return tuple(out)

            zero = jnp.zeros((_LANES,), jnp.float32)
            accs = lax.fori_loop(0, L // runroll, rbody, (zero,) * ncol)
            for c in range(ncol):
                pooled_v[b, pl.ds(c * _LANES, _LANES)] = accs[c] * inv_l

        for u in range(nbuf - 1):
            gather_bag(u, u)

        def body(q, carry):
            bq = nbuf * q
            for u in range(nbuf):
                b = bq + u
                nxt = b + nbuf - 1

                @pl.when(nxt < bpw)
                def _():
                    gather_bag(nxt, (u + nbuf - 1) % nbuf)

                drain_bag(b, u)
                reduce_bag(b, u)
            return carry

        lax.fori_loop(0, bpw // nbuf, body, 0)

        pltpu.sync_copy(pooled_v, out_hbm.at[pl.ds(base, bpw)])

    return pool(idx2, table)


def _mlp(pooled, W1, b1, W2, b2):
    B, E = pooled.shape
    H = W1.shape[1]
    C = W2.shape[1]
    W1T = W1.T
    W2T = W2.T
    cb = 600
    grid = (pl.cdiv(C, cb),)

    def body(p_ref, w1t_ref, b1_ref, w2t_ref, b2_ref, ot_ref):
        ht = jax.lax.dot_general(
            w1t_ref[...], p_ref[...], (((1,), (1,)), ((), ())),
            preferred_element_type=jnp.float32) + b1_ref[...]
        ht = jnp.maximum(ht, 0.0)
        ot = jnp.dot(w2t_ref[...], ht, preferred_element_type=jnp.float32)
        ot_ref[...] = ot + b2_ref[...]

    outT = pl.pallas_call(
        body,
        grid=grid,
        in_specs=[
            pl.BlockSpec((B, E), lambda j: (0, 0)),
            pl.BlockSpec((H, E), lambda j: (0, 0)),
            pl.BlockSpec((H, 1), lambda j: (0, 0)),
            pl.BlockSpec((cb, H), lambda j: (j, 0)),
            pl.BlockSpec((cb, 1), lambda j: (j, 0)),
        ],
        out_specs=pl.BlockSpec((cb, B), lambda j: (j, 0)),
        out_shape=jax.ShapeDtypeStruct((C, B), jnp.float32),
    )(pooled, W1T, b1.reshape(H, 1), W2T, b2.reshape(C, 1))
    return outT.T


def kernel(texts, table, W1, b1, W2, b2):
    B, L = texts.shape
    V, E = table.shape
    nchunk = 5
    k = L // nchunk
    packed = _linearize_table(table.T)
    rows2 = packed.shape[0] * 2
    table_p = packed.reshape(rows2, E)
    ti = texts.astype(jnp.int32)
    g_idx = ((ti >> 12) << 12) | ((ti & 2047) << 1) | ((ti >> 11) & 1)
    pooled = _pool(g_idx, table_p, nchunk=nchunk, k=k)
    return _mlp(pooled, W1, b1, W2, b2)
